# Initial kernel scaffold; baseline (speedup 1.0000x reference)
import jax
import jax.numpy as jnp
from jax.experimental import pallas as pl


def kernel(x, edge_index, W1, a_src1, a_dst1, b1, W2, a_src2, a_dst2, b2):
    raise NotImplementedError("write your pallas kernel here")


# speedup vs baseline: 46.6188x; 46.6188x over previous
import functools

import jax
import jax.numpy as jnp
from jax import lax
from jax.experimental import pallas as pl
from jax.experimental.pallas import tpu as pltpu
from jax.experimental.pallas import tpu_sc as plsc

N = 10000
E = 320000
D = 128
H = 4
PH = 32

NC = 2
NS = 16
NW = NC * NS
C = 128
CHUNKS_PW = 79
EPW = C * CHUNKS_PW
EPAD = NW * EPW
NP1 = N + 1
ROWS_PT = 640
NR = NS * ROWS_PT

BLK = 1000
NEG_SLOPE = 0.2


def _dense1_body(x_ref, w_ref, a8_ref, h_ref, al_ref, cm_ref):
    h = jnp.dot(x_ref[...], w_ref[...], preferred_element_type=jnp.float32)
    h_ref[...] = h
    al = jnp.dot(h, a8_ref[...], preferred_element_type=jnp.float32)
    al_ref[...] = al
    i = pl.program_id(0)

    @pl.when(i == 0)
    def _():
        cm_ref[...] = jnp.full((1, 8), -jnp.inf, jnp.float32)

    cm_ref[...] = jnp.maximum(cm_ref[...], jnp.max(al, axis=0, keepdims=True))


def _dense1(x, w1, a8):
    return pl.pallas_call(
        _dense1_body,
        grid=(N // BLK,),
        in_specs=[
            pl.BlockSpec((BLK, D), lambda i: (i, 0)),
            pl.BlockSpec((D, D), lambda i: (0, 0)),
            pl.BlockSpec((D, 8), lambda i: (0, 0)),
        ],
        out_specs=[
            pl.BlockSpec((BLK, D), lambda i: (i, 0)),
            pl.BlockSpec((BLK, 8), lambda i: (i, 0)),
            pl.BlockSpec((1, 8), lambda i: (0, 0)),
        ],
        out_shape=[
            jax.ShapeDtypeStruct((N, D), jnp.float32),
            jax.ShapeDtypeStruct((N, 8), jnp.float32),
            jax.ShapeDtypeStruct((1, 8), jnp.float32),
        ],
    )(x, w1, a8)


def _dense2_body(a0_ref, a1_ref, d0_ref, d1_ref, p_ref, b_ref, w_ref, a8_ref,
                 h_ref, al_ref, cm_ref):
    den = d0_ref[...] + d1_ref[...]
    rec = 1.0 / (den + 1e-16)
    rep = jnp.dot(rec, p_ref[...], preferred_element_type=jnp.float32)
    h1 = (a0_ref[...] + a1_ref[...]) * rep + b_ref[...]
    h1 = jnp.maximum(h1, 0.0)
    h2 = jnp.dot(h1, w_ref[...], preferred_element_type=jnp.float32)
    h_ref[...] = h2
    al = jnp.dot(h2, a8_ref[...], preferred_element_type=jnp.float32)
    al_ref[...] = al
    i = pl.program_id(0)

    @pl.when(i == 0)
    def _():
        cm_ref[...] = jnp.full((1, 8), -jnp.inf, jnp.float32)

    cm_ref[...] = jnp.maximum(cm_ref[...], jnp.max(al, axis=0, keepdims=True))


def _dense2(a0, a1, d0, d1, p16, b1, w2, a8):
    return pl.pallas_call(
        _dense2_body,
        grid=(N // BLK,),
        in_specs=[
            pl.BlockSpec((BLK, D), lambda i: (i, 0)),
            pl.BlockSpec((BLK, D), lambda i: (i, 0)),
            pl.BlockSpec((BLK, 16), lambda i: (i, 0)),
            pl.BlockSpec((BLK, 16), lambda i: (i, 0)),
            pl.BlockSpec((16, D), lambda i: (0, 0)),
            pl.BlockSpec((1, D), lambda i: (0, 0)),
            pl.BlockSpec((D, D), lambda i: (0, 0)),
            pl.BlockSpec((D, 8), lambda i: (0, 0)),
        ],
        out_specs=[
            pl.BlockSpec((BLK, D), lambda i: (i, 0)),
            pl.BlockSpec((BLK, 8), lambda i: (i, 0)),
            pl.BlockSpec((1, 8), lambda i: (0, 0)),
        ],
        out_shape=[
            jax.ShapeDtypeStruct((N, D), jnp.float32),
            jax.ShapeDtypeStruct((N, 8), jnp.float32),
            jax.ShapeDtypeStruct((1, 8), jnp.float32),
        ],
    )(a0, a1, d0, d1, p16, b1, w2, a8)


def _final_body(a0_ref, a1_ref, d0_ref, d1_ref, p_ref, b_ref, o_ref):
    den = d0_ref[...] + d1_ref[...]
    rec = 1.0 / (den + 1e-16)
    rep = jnp.dot(rec, p_ref[...], preferred_element_type=jnp.float32)
    o_ref[...] = (a0_ref[...] + a1_ref[...]) * rep + b_ref[...]


def _final(a0, a1, d0, d1, p16, b2):
    return pl.pallas_call(
        _final_body,
        grid=(N // BLK,),
        in_specs=[
            pl.BlockSpec((BLK, D), lambda i: (i, 0)),
            pl.BlockSpec((BLK, D), lambda i: (i, 0)),
            pl.BlockSpec((BLK, 16), lambda i: (i, 0)),
            pl.BlockSpec((BLK, 16), lambda i: (i, 0)),
            pl.BlockSpec((16, D), lambda i: (0, 0)),
            pl.BlockSpec((1, D), lambda i: (0, 0)),
        ],
        out_specs=pl.BlockSpec((BLK, D), lambda i: (i, 0)),
        out_shape=jax.ShapeDtypeStruct((N, D), jnp.float32),
    )(a0, a1, d0, d1, p16, b2)


def _edge_body(src_hbm, dst_hbm, h_hbm, al_hbm, m_hbm,
               acc_out, den_out,
               a_v, b_v, rows_v, wbuf_v, src_v, dst_v, m_v,
               acc_sp, den_sp, gsem, asem, bsem):
    cid = lax.axis_index("c")
    sid = lax.axis_index("s")
    wid = sid * NC + cid

    pltpu.sync_copy(m_hbm, m_v)
    mvec = m_v[:]
    ms = (mvec[0], mvec[1], mvec[2], mvec[3])

    zero16 = jnp.zeros((16,), jnp.float32)

    def _zrow(r, _):
        for k in range(8):
            rows_v[r, pl.ds(k * 16, 16)] = zero16
        wbuf_v[r, :] = zero16
        return 0

    lax.fori_loop(0, C, _zrow, 0)
    for j in range(ROWS_PT // C):
        r0 = sid * ROWS_PT + j * C
        pltpu.sync_copy(rows_v, acc_sp.at[pl.ds(r0, C)])
        pltpu.sync_copy(wbuf_v, den_sp.at[pl.ds(r0, C)])
    plsc.subcore_barrier()

    lanes = lax.iota(jnp.int32, 16)

    def _chunk(i, _):
        off = wid * EPW + i * C
        pltpu.sync_copy(src_hbm.at[pl.ds(off, C)], src_v)
        pltpu.sync_copy(dst_hbm.at[pl.ds(off, C)], dst_v)
        gather = pltpu.async_copy(h_hbm.at[src_v], rows_v, gsem)
        ga = pltpu.async_copy(al_hbm.at[src_v], a_v, asem)
        gb = pltpu.async_copy(al_hbm.at[dst_v], b_v, bsem)
        ga.wait()
        gb.wait()

        for g in range(C // 16):
            rows16 = lanes + g * 16
            for h in range(H):
                a = plsc.load_gather(a_v, [rows16, jnp.full((16,), h, jnp.int32)])
                b = plsc.load_gather(b_v, [rows16, jnp.full((16,), 4 + h, jnp.int32)])
                s = a + b
                e = jnp.where(s > 0, s, NEG_SLOPE * s)
                w = jnp.exp(e - ms[h])
                plsc.store_scatter(wbuf_v, [rows16, jnp.full((16,), h, jnp.int32)], w)
        pltpu.sync_copy(wbuf_v, den_sp.at[dst_v], add=True)
        gather.wait()

        def _scale(e, _):
            wv = wbuf_v[e, :]
            for h in range(H):
                s = wv[h]
                for q in range(2):
                    sl = pl.ds(h * PH + q * 16, 16)
                    rows_v[e, sl] = rows_v[e, sl] * s
            return 0

        lax.fori_loop(0, C, _scale, 0)
        pltpu.sync_copy(rows_v, acc_sp.at[dst_v], add=True)
        return 0

    lax.fori_loop(0, CHUNKS_PW, _chunk, 0)
    plsc.subcore_barrier()

    for j in range(ROWS_PT // C):
        r0 = sid * ROWS_PT + j * C
        pltpu.sync_copy(acc_sp.at[pl.ds(r0, C)], rows_v)
        pltpu.sync_copy(rows_v, acc_out.at[cid, pl.ds(r0, C)])
        pltpu.sync_copy(den_sp.at[pl.ds(r0, C)], wbuf_v)
        pltpu.sync_copy(wbuf_v, den_out.at[cid, pl.ds(r0, C)])


_edge_pass = functools.partial(
    pl.kernel,
    out_type=[
        jax.ShapeDtypeStruct((NC, NR, D), jnp.float32),
        jax.ShapeDtypeStruct((NC, NR, 16), jnp.float32),
    ],
    mesh=plsc.VectorSubcoreMesh(
        core_axis_name="c", subcore_axis_name="s", num_cores=NC, num_subcores=NS),
    compiler_params=pltpu.CompilerParams(
        needs_layout_passes=False, use_tc_tiling_on_sc=False),
    scratch_types=[
        pltpu.VMEM((C, 16), jnp.float32),
        pltpu.VMEM((C, 16), jnp.float32),
        pltpu.VMEM((C, D), jnp.float32),
        pltpu.VMEM((C, 16), jnp.float32),
        pltpu.VMEM((C,), jnp.int32),
        pltpu.VMEM((C,), jnp.int32),
        pltpu.VMEM((16,), jnp.float32),
        pltpu.VMEM_SHARED((NR, D), jnp.float32),
        pltpu.VMEM_SHARED((NR, 16), jnp.float32),
        pltpu.SemaphoreType.DMA,
        pltpu.SemaphoreType.DMA,
        pltpu.SemaphoreType.DMA,
    ],
)(_edge_body)


def _layer_edge_pass(srcp, dstp, h, al, cm):
    hp = jnp.concatenate([h, jnp.zeros((1, D), jnp.float32)], axis=0)
    alp = jnp.concatenate(
        [jnp.concatenate([al, jnp.zeros((1, 8), jnp.float32)], axis=0),
         jnp.zeros((NP1, 8), jnp.float32)], axis=1)
    m4 = jnp.maximum(cm[0, :4] + cm[0, 4:], 0.0)
    m16 = jnp.concatenate([m4, jnp.zeros((12,), jnp.float32)])
    accs, dens = _edge_pass(srcp, dstp, hp, alp, m16)
    return (accs[0, :N], accs[1, :N], dens[0, :N], dens[1, :N])


def kernel(x, edge_index, W1, a_src1, a_dst1, b1, W2, a_src2, a_dst2, b2):
    src = edge_index[0].astype(jnp.int32)
    dst = edge_index[1].astype(jnp.int32)
    pad = jnp.full((EPAD - E,), N, jnp.int32)
    srcp = jnp.concatenate([src, pad])
    dstp = jnp.concatenate([dst, pad])

    eye4 = jnp.eye(4, dtype=jnp.float32)
    a8_1 = jnp.concatenate(
        [(a_src1[:, :, None] * eye4[:, None, :]).reshape(D, H),
         (a_dst1[:, :, None] * eye4[:, None, :]).reshape(D, H)], axis=1)
    a8_2 = jnp.concatenate(
        [(a_src2[:, :, None] * eye4[:, None, :]).reshape(D, H),
         (a_dst2[:, :, None] * eye4[:, None, :]).reshape(D, H)], axis=1)
    p16 = jnp.concatenate(
        [jnp.repeat(eye4, PH, axis=1), jnp.zeros((12, D), jnp.float32)], axis=0)

    h1, al1, cm1 = _dense1(x, W1, a8_1)
    a0, a1, d0, d1 = _layer_edge_pass(srcp, dstp, h1, al1, cm1)
    h2, al2, cm2 = _dense2(a0, a1, d0, d1, p16, b1.reshape(1, D), W2, a8_2)
    a0, a1, d0, d1 = _layer_edge_pass(srcp, dstp, h2, al2, cm2)
    return _final(a0, a1, d0, d1, p16, b2.reshape(1, D))


# speedup vs baseline: 55.5965x; 1.1926x over previous
import functools

import jax
import jax.numpy as jnp
from jax import lax
from jax.experimental import pallas as pl
from jax.experimental.pallas import tpu as pltpu
from jax.experimental.pallas import tpu_sc as plsc

N = 10000
E = 320000
D = 128
H = 4
PH = 32

NC = 2
NS = 16
NW = NC * NS
C = 128
CHUNKS_PW = 79
EPW = C * CHUNKS_PW
EPAD = NW * EPW
NP1 = N + 1
ROWS_PT = 628
NR = NS * ROWS_PT
WB = (128, 128, 128, 128, 116)

BLK = 1000
NEG_SLOPE = 0.2


def _dense1_body(x_ref, w_ref, a8_ref, h_ref, al_ref, cm_ref):
    h = jnp.dot(x_ref[...], w_ref[...], preferred_element_type=jnp.float32)
    h_ref[...] = h
    al = jnp.dot(h, a8_ref[...], preferred_element_type=jnp.float32)
    al_ref[...] = al
    i = pl.program_id(0)

    @pl.when(i == 0)
    def _():
        cm_ref[...] = jnp.full((1, 8), -jnp.inf, jnp.float32)

    cm_ref[...] = jnp.maximum(cm_ref[...], jnp.max(al, axis=0, keepdims=True))


def _dense1(x, w1, a8):
    return pl.pallas_call(
        _dense1_body,
        grid=(N // BLK,),
        in_specs=[
            pl.BlockSpec((BLK, D), lambda i: (i, 0)),
            pl.BlockSpec((D, D), lambda i: (0, 0)),
            pl.BlockSpec((D, 8), lambda i: (0, 0)),
        ],
        out_specs=[
            pl.BlockSpec((BLK, D), lambda i: (i, 0)),
            pl.BlockSpec((BLK, 8), lambda i: (i, 0)),
            pl.BlockSpec((1, 8), lambda i: (0, 0)),
        ],
        out_shape=[
            jax.ShapeDtypeStruct((N, D), jnp.float32),
            jax.ShapeDtypeStruct((N, 8), jnp.float32),
            jax.ShapeDtypeStruct((1, 8), jnp.float32),
        ],
    )(x, w1, a8)


def _dense2_body(a0_ref, a1_ref, d0_ref, d1_ref, p_ref, b_ref, w_ref, a8_ref,
                 h_ref, al_ref, cm_ref):
    den = d0_ref[...] + d1_ref[...]
    rec = 1.0 / (den + 1e-16)
    rep = jnp.dot(rec, p_ref[...], preferred_element_type=jnp.float32)
    h1 = (a0_ref[...] + a1_ref[...]) * rep + b_ref[...]
    h1 = jnp.maximum(h1, 0.0)
    h2 = jnp.dot(h1, w_ref[...], preferred_element_type=jnp.float32)
    h_ref[...] = h2
    al = jnp.dot(h2, a8_ref[...], preferred_element_type=jnp.float32)
    al_ref[...] = al
    i = pl.program_id(0)

    @pl.when(i == 0)
    def _():
        cm_ref[...] = jnp.full((1, 8), -jnp.inf, jnp.float32)

    cm_ref[...] = jnp.maximum(cm_ref[...], jnp.max(al, axis=0, keepdims=True))


def _dense2(a0, a1, d0, d1, p16, b1, w2, a8):
    return pl.pallas_call(
        _dense2_body,
        grid=(N // BLK,),
        in_specs=[
            pl.BlockSpec((BLK, D), lambda i: (i, 0)),
            pl.BlockSpec((BLK, D), lambda i: (i, 0)),
            pl.BlockSpec((BLK, 16), lambda i: (i, 0)),
            pl.BlockSpec((BLK, 16), lambda i: (i, 0)),
            pl.BlockSpec((16, D), lambda i: (0, 0)),
            pl.BlockSpec((1, D), lambda i: (0, 0)),
            pl.BlockSpec((D, D), lambda i: (0, 0)),
            pl.BlockSpec((D, 8), lambda i: (0, 0)),
        ],
        out_specs=[
            pl.BlockSpec((BLK, D), lambda i: (i, 0)),
            pl.BlockSpec((BLK, 8), lambda i: (i, 0)),
            pl.BlockSpec((1, 8), lambda i: (0, 0)),
        ],
        out_shape=[
            jax.ShapeDtypeStruct((N, D), jnp.float32),
            jax.ShapeDtypeStruct((N, 8), jnp.float32),
            jax.ShapeDtypeStruct((1, 8), jnp.float32),
        ],
    )(a0, a1, d0, d1, p16, b1, w2, a8)


def _final_body(a0_ref, a1_ref, d0_ref, d1_ref, p_ref, b_ref, o_ref):
    den = d0_ref[...] + d1_ref[...]
    rec = 1.0 / (den + 1e-16)
    rep = jnp.dot(rec, p_ref[...], preferred_element_type=jnp.float32)
    o_ref[...] = (a0_ref[...] + a1_ref[...]) * rep + b_ref[...]


def _final(a0, a1, d0, d1, p16, b2):
    return pl.pallas_call(
        _final_body,
        grid=(N // BLK,),
        in_specs=[
            pl.BlockSpec((BLK, D), lambda i: (i, 0)),
            pl.BlockSpec((BLK, D), lambda i: (i, 0)),
            pl.BlockSpec((BLK, 16), lambda i: (i, 0)),
            pl.BlockSpec((BLK, 16), lambda i: (i, 0)),
            pl.BlockSpec((16, D), lambda i: (0, 0)),
            pl.BlockSpec((1, D), lambda i: (0, 0)),
        ],
        out_specs=pl.BlockSpec((BLK, D), lambda i: (i, 0)),
        out_shape=jax.ShapeDtypeStruct((N, D), jnp.float32),
    )(a0, a1, d0, d1, p16, b2)


def _edge_body(src_hbm, dst_hbm, h_hbm, al_hbm, m_hbm,
               acc_out, den_out,
               a_v, b_v, rows0_v, rows1_v, wbuf_v,
               src0_v, src1_v, dst0_v, dst1_v, m_v,
               acc_sp, den_sp, gsem0, gsem1, asem, bsem):
    cid = lax.axis_index("c")
    sid = lax.axis_index("s")
    wid = sid * NC + cid
    rows = (rows0_v, rows1_v)
    srcs = (src0_v, src1_v)
    dsts = (dst0_v, dst1_v)
    gsems = (gsem0, gsem1)

    pltpu.sync_copy(m_hbm, m_v)
    mvec = m_v[:]
    ms = (mvec[0], mvec[1], mvec[2], mvec[3])

    zero16 = jnp.zeros((16,), jnp.float32)

    def _zrow(r, _):
        for k in range(8):
            rows0_v[r, pl.ds(k * 16, 16)] = zero16
        wbuf_v[r, :] = zero16
        return 0

    lax.fori_loop(0, C, _zrow, 0)
    r0 = sid * ROWS_PT
    for n in WB:
        pltpu.sync_copy(rows0_v.at[pl.ds(0, n)], acc_sp.at[pl.ds(r0, n)])
        pltpu.sync_copy(wbuf_v.at[pl.ds(0, n)], den_sp.at[pl.ds(r0, n)])
        r0 += n
    plsc.subcore_barrier()

    lanes = lax.iota(jnp.int32, 16)
    base = wid * EPW

    def _fire(c, p):
        off = base + c * C
        pltpu.sync_copy(src_hbm.at[pl.ds(off, C)], srcs[p])
        pltpu.sync_copy(dst_hbm.at[pl.ds(off, C)], dsts[p])
        return pltpu.async_copy(h_hbm.at[srcs[p]], rows[p], gsems[p])

    def _process(p, gather):
        ga = pltpu.async_copy(al_hbm.at[srcs[p]], a_v, asem)
        gb = pltpu.async_copy(al_hbm.at[dsts[p]], b_v, bsem)
        ga.wait()
        gb.wait()
        rows_v = rows[p]

        for g in range(C // 16):
            rows16 = lanes + g * 16
            for h in range(H):
                a = plsc.load_gather(a_v, [rows16, jnp.full((16,), h, jnp.int32)])
                b = plsc.load_gather(b_v, [rows16, jnp.full((16,), 4 + h, jnp.int32)])
                s = a + b
                e = jnp.where(s > 0, s, NEG_SLOPE * s)
                w = jnp.exp(e - ms[h])
                plsc.store_scatter(wbuf_v, [rows16, jnp.full((16,), h, jnp.int32)], w)
        pltpu.sync_copy(wbuf_v, den_sp.at[dsts[p]], add=True)
        gather.wait()

        @plsc.parallel_loop(0, C, unroll=2)
        def _scale(e):
            wv = wbuf_v[e, :]
            for h in range(H):
                s = wv[h]
                for q in range(2):
                    sl = pl.ds(h * PH + q * 16, 16)
                    rows_v[e, sl] = rows_v[e, sl] * s

        pltpu.sync_copy(rows_v, acc_sp.at[dsts[p]], add=True)

    _fire(0, 0)

    def _pair_body(i, _):
        c0 = 2 * i
        g_even = pltpu.make_async_copy(h_hbm.at[srcs[0]], rows[0], gsems[0])
        g_odd = _fire(c0 + 1, 1)
        _process(0, g_even)
        _fire(c0 + 2, 0)
        _process(1, g_odd)
        return 0

    lax.fori_loop(0, (CHUNKS_PW - 1) // 2, _pair_body, 0)
    _process(0, pltpu.make_async_copy(h_hbm.at[srcs[0]], rows[0], gsems[0]))
    plsc.subcore_barrier()

    r0 = sid * ROWS_PT
    for n in WB:
        pltpu.sync_copy(acc_sp.at[pl.ds(r0, n)], rows0_v.at[pl.ds(0, n)])
        pltpu.sync_copy(rows0_v.at[pl.ds(0, n)], acc_out.at[cid, pl.ds(r0, n)])
        pltpu.sync_copy(den_sp.at[pl.ds(r0, n)], wbuf_v.at[pl.ds(0, n)])
        pltpu.sync_copy(wbuf_v.at[pl.ds(0, n)], den_out.at[cid, pl.ds(r0, n)])
        r0 += n


_edge_pass = functools.partial(
    pl.kernel,
    out_type=[
        jax.ShapeDtypeStruct((NC, NR, D), jnp.float32),
        jax.ShapeDtypeStruct((NC, NR, 16), jnp.float32),
    ],
    mesh=plsc.VectorSubcoreMesh(
        core_axis_name="c", subcore_axis_name="s", num_cores=NC, num_subcores=NS),
    compiler_params=pltpu.CompilerParams(
        needs_layout_passes=False, use_tc_tiling_on_sc=False),
    scratch_types=[
        pltpu.VMEM((C, 16), jnp.float32),
        pltpu.VMEM((C, 16), jnp.float32),
        pltpu.VMEM((C, D), jnp.float32),
        pltpu.VMEM((C, D), jnp.float32),
        pltpu.VMEM((C, 16), jnp.float32),
        pltpu.VMEM((C,), jnp.int32),
        pltpu.VMEM((C,), jnp.int32),
        pltpu.VMEM((C,), jnp.int32),
        pltpu.VMEM((C,), jnp.int32),
        pltpu.VMEM((16,), jnp.float32),
        pltpu.VMEM_SHARED((NR, D), jnp.float32),
        pltpu.VMEM_SHARED((NR, 16), jnp.float32),
        pltpu.SemaphoreType.DMA,
        pltpu.SemaphoreType.DMA,
        pltpu.SemaphoreType.DMA,
        pltpu.SemaphoreType.DMA,
    ],
)(_edge_body)


def _layer_edge_pass(srcp, dstp, h, al, cm):
    hp = jnp.concatenate([h, jnp.zeros((1, D), jnp.float32)], axis=0)
    alp = jnp.concatenate(
        [jnp.concatenate([al, jnp.zeros((1, 8), jnp.float32)], axis=0),
         jnp.zeros((NP1, 8), jnp.float32)], axis=1)
    m4 = jnp.maximum(cm[0, :4] + cm[0, 4:], 0.0)
    m16 = jnp.concatenate([m4, jnp.zeros((12,), jnp.float32)])
    accs, dens = _edge_pass(srcp, dstp, hp, alp, m16)
    return (accs[0, :N], accs[1, :N], dens[0, :N], dens[1, :N])


def kernel(x, edge_index, W1, a_src1, a_dst1, b1, W2, a_src2, a_dst2, b2):
    src = edge_index[0].astype(jnp.int32)
    dst = edge_index[1].astype(jnp.int32)
    pad = jnp.full((EPAD - E,), N, jnp.int32)
    srcp = jnp.concatenate([src, pad])
    dstp = jnp.concatenate([dst, pad])

    eye4 = jnp.eye(4, dtype=jnp.float32)
    a8_1 = jnp.concatenate(
        [(a_src1[:, :, None] * eye4[:, None, :]).reshape(D, H),
         (a_dst1[:, :, None] * eye4[:, None, :]).reshape(D, H)], axis=1)
    a8_2 = jnp.concatenate(
        [(a_src2[:, :, None] * eye4[:, None, :]).reshape(D, H),
         (a_dst2[:, :, None] * eye4[:, None, :]).reshape(D, H)], axis=1)
    p16 = jnp.concatenate(
        [jnp.repeat(eye4, PH, axis=1), jnp.zeros((12, D), jnp.float32)], axis=0)

    h1, al1, cm1 = _dense1(x, W1, a8_1)
    a0, a1, d0, d1 = _layer_edge_pass(srcp, dstp, h1, al1, cm1)
    h2, al2, cm2 = _dense2(a0, a1, d0, d1, p16, b1.reshape(1, D), W2, a8_2)
    a0, a1, d0, d1 = _layer_edge_pass(srcp, dstp, h2, al2, cm2)
    return _final(a0, a1, d0, d1, p16, b2.reshape(1, D))


# speedup vs baseline: 56.6458x; 1.0189x over previous
import functools

import jax
import jax.numpy as jnp
from jax import lax
from jax.experimental import pallas as pl
from jax.experimental.pallas import tpu as pltpu
from jax.experimental.pallas import tpu_sc as plsc

N = 10000
E = 320000
D = 128
H = 4
PH = 32

NC = 2
NS = 16
NW = NC * NS
C = 128
CHUNKS_PW = 79
EPW = C * CHUNKS_PW
EPAD = NW * EPW
NP1 = N + 1
ROWS_PT = 628
NR = NS * ROWS_PT
WB = (128, 128, 128, 128, 116)

BLK = 1000
NEG_SLOPE = 0.2


def _dense1_body(x_ref, w_ref, a8_ref, h_ref, al_ref, cm_ref):
    h = jnp.dot(x_ref[...], w_ref[...], preferred_element_type=jnp.float32)
    h_ref[...] = h
    al = jnp.dot(h, a8_ref[...], preferred_element_type=jnp.float32)
    al_ref[...] = al
    i = pl.program_id(0)

    @pl.when(i == 0)
    def _():
        cm_ref[...] = jnp.full((1, 8), -jnp.inf, jnp.float32)

    cm_ref[...] = jnp.maximum(cm_ref[...], jnp.max(al, axis=0, keepdims=True))


def _dense1(x, w1, a8):
    return pl.pallas_call(
        _dense1_body,
        grid=(N // BLK,),
        in_specs=[
            pl.BlockSpec((BLK, D), lambda i: (i, 0)),
            pl.BlockSpec((D, D), lambda i: (0, 0)),
            pl.BlockSpec((D, 8), lambda i: (0, 0)),
        ],
        out_specs=[
            pl.BlockSpec((BLK, D), lambda i: (i, 0)),
            pl.BlockSpec((BLK, 8), lambda i: (i, 0)),
            pl.BlockSpec((1, 8), lambda i: (0, 0)),
        ],
        out_shape=[
            jax.ShapeDtypeStruct((N, D), jnp.float32),
            jax.ShapeDtypeStruct((N, 8), jnp.float32),
            jax.ShapeDtypeStruct((1, 8), jnp.float32),
        ],
    )(x, w1, a8)


def _dense2_body(a0_ref, a1_ref, d0_ref, d1_ref, p_ref, b_ref, w_ref, a8_ref,
                 h_ref, al_ref, cm_ref):
    den = d0_ref[...] + d1_ref[...]
    rec = 1.0 / (den + 1e-16)
    rep = jnp.dot(rec, p_ref[...], preferred_element_type=jnp.float32)
    h1 = (a0_ref[...] + a1_ref[...]) * rep + b_ref[...]
    h1 = jnp.maximum(h1, 0.0)
    h2 = jnp.dot(h1, w_ref[...], preferred_element_type=jnp.float32)
    h_ref[...] = h2
    al = jnp.dot(h2, a8_ref[...], preferred_element_type=jnp.float32)
    al_ref[...] = al
    i = pl.program_id(0)

    @pl.when(i == 0)
    def _():
        cm_ref[...] = jnp.full((1, 8), -jnp.inf, jnp.float32)

    cm_ref[...] = jnp.maximum(cm_ref[...], jnp.max(al, axis=0, keepdims=True))


def _dense2(a0, a1, d0, d1, p16, b1, w2, a8):
    return pl.pallas_call(
        _dense2_body,
        grid=(N // BLK,),
        in_specs=[
            pl.BlockSpec((BLK, D), lambda i: (i, 0)),
            pl.BlockSpec((BLK, D), lambda i: (i, 0)),
            pl.BlockSpec((BLK, 16), lambda i: (i, 0)),
            pl.BlockSpec((BLK, 16), lambda i: (i, 0)),
            pl.BlockSpec((16, D), lambda i: (0, 0)),
            pl.BlockSpec((1, D), lambda i: (0, 0)),
            pl.BlockSpec((D, D), lambda i: (0, 0)),
            pl.BlockSpec((D, 8), lambda i: (0, 0)),
        ],
        out_specs=[
            pl.BlockSpec((BLK, D), lambda i: (i, 0)),
            pl.BlockSpec((BLK, 8), lambda i: (i, 0)),
            pl.BlockSpec((1, 8), lambda i: (0, 0)),
        ],
        out_shape=[
            jax.ShapeDtypeStruct((N, D), jnp.float32),
            jax.ShapeDtypeStruct((N, 8), jnp.float32),
            jax.ShapeDtypeStruct((1, 8), jnp.float32),
        ],
    )(a0, a1, d0, d1, p16, b1, w2, a8)


def _final_body(a0_ref, a1_ref, d0_ref, d1_ref, p_ref, b_ref, o_ref):
    den = d0_ref[...] + d1_ref[...]
    rec = 1.0 / (den + 1e-16)
    rep = jnp.dot(rec, p_ref[...], preferred_element_type=jnp.float32)
    o_ref[...] = (a0_ref[...] + a1_ref[...]) * rep + b_ref[...]


def _final(a0, a1, d0, d1, p16, b2):
    return pl.pallas_call(
        _final_body,
        grid=(N // BLK,),
        in_specs=[
            pl.BlockSpec((BLK, D), lambda i: (i, 0)),
            pl.BlockSpec((BLK, D), lambda i: (i, 0)),
            pl.BlockSpec((BLK, 16), lambda i: (i, 0)),
            pl.BlockSpec((BLK, 16), lambda i: (i, 0)),
            pl.BlockSpec((16, D), lambda i: (0, 0)),
            pl.BlockSpec((1, D), lambda i: (0, 0)),
        ],
        out_specs=pl.BlockSpec((BLK, D), lambda i: (i, 0)),
        out_shape=jax.ShapeDtypeStruct((N, D), jnp.float32),
    )(a0, a1, d0, d1, p16, b2)


def _edge_body(src_hbm, dst_hbm, h_hbm, al_hbm, m_hbm,
               acc_out, den_out,
               a_v, b_v, rows0_v, rows1_v, wbuf_v,
               src0_v, src1_v, dst0_v, dst1_v, m_v,
               acc_sp, den_sp, gsem0, gsem1, asem, bsem, dsem):
    cid = lax.axis_index("c")
    sid = lax.axis_index("s")
    wid = sid * NC + cid
    rows = (rows0_v, rows1_v)
    srcs = (src0_v, src1_v)
    dsts = (dst0_v, dst1_v)
    gsems = (gsem0, gsem1)

    pltpu.sync_copy(m_hbm, m_v)
    mvec = m_v[:]
    ms = (mvec[0], mvec[1], mvec[2], mvec[3])

    zero16 = jnp.zeros((16,), jnp.float32)

    def _zrow(r, _):
        for k in range(8):
            rows0_v[r, pl.ds(k * 16, 16)] = zero16
        wbuf_v[r, :] = zero16
        return 0

    lax.fori_loop(0, C, _zrow, 0)
    r0 = sid * ROWS_PT
    for n in WB:
        pltpu.sync_copy(rows0_v.at[pl.ds(0, n)], acc_sp.at[pl.ds(r0, n)])
        pltpu.sync_copy(wbuf_v.at[pl.ds(0, n)], den_sp.at[pl.ds(r0, n)])
        r0 += n
    plsc.subcore_barrier()

    lanes = lax.iota(jnp.int32, 16)
    base = wid * EPW

    def _fire(c, p):
        off = base + c * C
        pltpu.sync_copy(src_hbm.at[pl.ds(off, C)], srcs[p])
        pltpu.sync_copy(dst_hbm.at[pl.ds(off, C)], dsts[p])
        return pltpu.async_copy(h_hbm.at[srcs[p]], rows[p], gsems[p])

    def _process(p, gather):
        ga = pltpu.async_copy(al_hbm.at[srcs[p]], a_v, asem)
        gb = pltpu.async_copy(al_hbm.at[dsts[p]], b_v, bsem)
        ga.wait()
        gb.wait()
        rows_v = rows[p]

        for g in range(C // 16):
            rows16 = lanes + g * 16
            for h in range(H):
                a = plsc.load_gather(a_v, [rows16, jnp.full((16,), h, jnp.int32)])
                b = plsc.load_gather(b_v, [rows16, jnp.full((16,), 4 + h, jnp.int32)])
                s = a + b
                e = jnp.where(s > 0, s, NEG_SLOPE * s)
                w = jnp.exp(e - ms[h])
                plsc.store_scatter(wbuf_v, [rows16, jnp.full((16,), h, jnp.int32)], w)
        den_add = pltpu.async_copy(wbuf_v, den_sp.at[dsts[p]], dsem, add=True)
        gather.wait()

        @plsc.parallel_loop(0, C, unroll=4)
        def _scale(e):
            wv = wbuf_v[e, :]
            for h in range(H):
                s = wv[h]
                for q in range(2):
                    sl = pl.ds(h * PH + q * 16, 16)
                    rows_v[e, sl] = rows_v[e, sl] * s

        pltpu.sync_copy(rows_v, acc_sp.at[dsts[p]], add=True)
        den_add.wait()

    _fire(0, 0)

    def _pair_body(i, _):
        c0 = 2 * i
        g_even = pltpu.make_async_copy(h_hbm.at[srcs[0]], rows[0], gsems[0])
        g_odd = _fire(c0 + 1, 1)
        _process(0, g_even)
        _fire(c0 + 2, 0)
        _process(1, g_odd)
        return 0

    lax.fori_loop(0, (CHUNKS_PW - 1) // 2, _pair_body, 0)
    _process(0, pltpu.make_async_copy(h_hbm.at[srcs[0]], rows[0], gsems[0]))
    plsc.subcore_barrier()

    r0 = sid * ROWS_PT
    for n in WB:
        pltpu.sync_copy(acc_sp.at[pl.ds(r0, n)], rows0_v.at[pl.ds(0, n)])
        pltpu.sync_copy(rows0_v.at[pl.ds(0, n)], acc_out.at[cid, pl.ds(r0, n)])
        pltpu.sync_copy(den_sp.at[pl.ds(r0, n)], wbuf_v.at[pl.ds(0, n)])
        pltpu.sync_copy(wbuf_v.at[pl.ds(0, n)], den_out.at[cid, pl.ds(r0, n)])
        r0 += n


_edge_pass = functools.partial(
    pl.kernel,
    out_type=[
        jax.ShapeDtypeStruct((NC, NR, D), jnp.float32),
        jax.ShapeDtypeStruct((NC, NR, 16), jnp.float32),
    ],
    mesh=plsc.VectorSubcoreMesh(
        core_axis_name="c", subcore_axis_name="s", num_cores=NC, num_subcores=NS),
    compiler_params=pltpu.CompilerParams(
        needs_layout_passes=False, use_tc_tiling_on_sc=False),
    scratch_types=[
        pltpu.VMEM((C, 16), jnp.float32),
        pltpu.VMEM((C, 16), jnp.float32),
        pltpu.VMEM((C, D), jnp.float32),
        pltpu.VMEM((C, D), jnp.float32),
        pltpu.VMEM((C, 16), jnp.float32),
        pltpu.VMEM((C,), jnp.int32),
        pltpu.VMEM((C,), jnp.int32),
        pltpu.VMEM((C,), jnp.int32),
        pltpu.VMEM((C,), jnp.int32),
        pltpu.VMEM((16,), jnp.float32),
        pltpu.VMEM_SHARED((NR, D), jnp.float32),
        pltpu.VMEM_SHARED((NR, 16), jnp.float32),
        pltpu.SemaphoreType.DMA,
        pltpu.SemaphoreType.DMA,
        pltpu.SemaphoreType.DMA,
        pltpu.SemaphoreType.DMA,
        pltpu.SemaphoreType.DMA,
    ],
)(_edge_body)


def _layer_edge_pass(srcp, dstp, h, al, cm):
    hp = jnp.concatenate([h, jnp.zeros((1, D), jnp.float32)], axis=0)
    alp = jnp.concatenate(
        [jnp.concatenate([al, jnp.zeros((1, 8), jnp.float32)], axis=0),
         jnp.zeros((NP1, 8), jnp.float32)], axis=1)
    m4 = jnp.maximum(cm[0, :4] + cm[0, 4:], 0.0)
    m16 = jnp.concatenate([m4, jnp.zeros((12,), jnp.float32)])
    accs, dens = _edge_pass(srcp, dstp, hp, alp, m16)
    return (accs[0, :N], accs[1, :N], dens[0, :N], dens[1, :N])


def kernel(x, edge_index, W1, a_src1, a_dst1, b1, W2, a_src2, a_dst2, b2):
    src = edge_index[0].astype(jnp.int32)
    dst = edge_index[1].astype(jnp.int32)
    pad = jnp.full((EPAD - E,), N, jnp.int32)
    srcp = jnp.concatenate([src, pad])
    dstp = jnp.concatenate([dst, pad])

    eye4 = jnp.eye(4, dtype=jnp.float32)
    a8_1 = jnp.concatenate(
        [(a_src1[:, :, None] * eye4[:, None, :]).reshape(D, H),
         (a_dst1[:, :, None] * eye4[:, None, :]).reshape(D, H)], axis=1)
    a8_2 = jnp.concatenate(
        [(a_src2[:, :, None] * eye4[:, None, :]).reshape(D, H),
         (a_dst2[:, :, None] * eye4[:, None, :]).reshape(D, H)], axis=1)
    p16 = jnp.concatenate(
        [jnp.repeat(eye4, PH, axis=1), jnp.zeros((12, D), jnp.float32)], axis=0)

    h1, al1, cm1 = _dense1(x, W1, a8_1)
    a0, a1, d0, d1 = _layer_edge_pass(srcp, dstp, h1, al1, cm1)
    h2, al2, cm2 = _dense2(a0, a1, d0, d1, p16, b1.reshape(1, D), W2, a8_2)
    a0, a1, d0, d1 = _layer_edge_pass(srcp, dstp, h2, al2, cm2)
    return _final(a0, a1, d0, d1, p16, b2.reshape(1, D))
